# baseline (speedup 1.0000x reference)
import functools

import jax
import jax.numpy as jnp
from jax import lax
from jax.experimental import pallas as pl
from jax.experimental.pallas import tpu as pltpu
from jax.experimental.pallas import tpu_sc as plsc

_B = 4096
_D = 16
_NC = 2
_NS = 16
_NW = _NC * _NS
_RPW = _B // _NW
_EPS = 1e-10
_DIS_PEN = 0.01
_INT_W = 0.1
_POP_W = 0.1
_KL_W = 0.01


def _sc_gather_body(u_int_tab, u_pop_tab, i_int_tab, i_pop_tab,
                    uidx_hbm, pidx_hbm, nidx_hbm,
                    u_int, u_pop, p_int, p_pop, n_int, n_pop,
                    uidx_v, pidx_v, nidx_v,
                    r0, r1, r2, r3, r4, r5, sem):
    wid = lax.axis_index("s") * _NC + lax.axis_index("c")
    base = wid * _RPW
    sl = pl.ds(base, _RPW)
    pltpu.sync_copy(uidx_hbm.at[sl], uidx_v)
    pltpu.sync_copy(pidx_hbm.at[sl], pidx_v)
    pltpu.sync_copy(nidx_hbm.at[sl], nidx_v)
    plan = (
        (u_int_tab, uidx_v, r0, u_int),
        (u_pop_tab, uidx_v, r1, u_pop),
        (i_int_tab, pidx_v, r2, p_int),
        (i_pop_tab, pidx_v, r3, p_pop),
        (i_int_tab, nidx_v, r4, n_int),
        (i_pop_tab, nidx_v, r5, n_pop),
    )
    descs = [pltpu.async_copy(tab.at[idx], buf, sem)
             for tab, idx, buf, _ in plan]
    for d, (_, _, buf, out) in zip(descs, plan):
        d.wait()
        pltpu.sync_copy(buf, out.at[sl])


@functools.cache
def _sc_gather():
    return pl.kernel(
        _sc_gather_body,
        out_type=tuple(jax.ShapeDtypeStruct((_B, _D), jnp.float32)
                       for _ in range(6)),
        mesh=plsc.VectorSubcoreMesh(core_axis_name="c", subcore_axis_name="s",
                                    num_cores=_NC, num_subcores=_NS),
        scratch_types=(
            [pltpu.VMEM((_RPW,), jnp.int32)] * 3
            + [pltpu.VMEM((_RPW, _D), jnp.float32)] * 6
            + [pltpu.SemaphoreType.DMA]
        ),
        compiler_params=pltpu.CompilerParams(use_tc_tiling_on_sc=False),
    )


def _gram(a, b):
    return lax.dot_general(a, b, (((0,), (0,)), ((), ())),
                           preferred_element_type=jnp.float32,
                           precision=lax.Precision.HIGHEST)


def _logsig(x):
    return jnp.log(1.0 / (1.0 + jnp.exp(-x)) + _EPS)


def _tc_counts_body(iidx_row, iidx_col, uidx_row, uidx_col, wi_ref, wu_ref):
    chunk = 256

    def item_chunk(j, _):
        rows = iidx_col[pl.ds(j * chunk, chunk), :]
        eq = (rows == iidx_row[:]).astype(jnp.float32)
        wi_ref[pl.ds(j * chunk, chunk), :] = 1.0 / jnp.sum(eq, axis=1, keepdims=True)
        return 0

    def user_chunk(j, _):
        rows = uidx_col[pl.ds(j * chunk, chunk), :]
        eq = (rows == uidx_row[:]).astype(jnp.float32)
        wu_ref[pl.ds(j * chunk, chunk), :] = 1.0 / jnp.sum(eq, axis=1, keepdims=True)
        return 0

    lax.fori_loop(0, (2 * _B) // chunk, item_chunk, 0)
    lax.fori_loop(0, _B // chunk, user_chunk, 0)


_tc_counts = pl.pallas_call(
    _tc_counts_body,
    out_shape=[jax.ShapeDtypeStruct((2 * _B, 1), jnp.float32),
               jax.ShapeDtypeStruct((_B, 1), jnp.float32)],
)


def _tc_loss_body(u_int, u_pop, p_int, p_pop, n_int, n_pop,
                  wi_ref, wu_ref, maskf, out_ref):
    wi = wi_ref[:]
    wp = wi[:_B]
    wn = wi[_B:]
    wu = wu_ref[:]
    n_item = jnp.sum(wi)
    n_user = jnp.sum(wu)

    ui = u_int[:]
    up = u_pop[:]
    pi = p_int[:]
    pp = p_pop[:]
    ni = n_int[:]
    npp = n_pop[:]

    ps_int = jnp.sum(ui * pi, axis=1, keepdims=True)
    ns_int = jnp.sum(ui * ni, axis=1, keepdims=True)
    ps_pop = jnp.sum(up * pp, axis=1, keepdims=True)
    ns_pop = jnp.sum(up * npp, axis=1, keepdims=True)
    mf = maskf[:]
    imf = 1.0 - mf
    loss_int = -jnp.mean(mf * _logsig(ps_int - ns_int))
    loss_pop = (-jnp.mean(mf * _logsig(ns_pop - ps_pop))
                - jnp.mean(imf * _logsig(ps_pop - ns_pop)))
    loss_total = -jnp.mean(_logsig((ps_int + ps_pop) - (ns_int + ns_pop)))

    d_items = jnp.sum(wp * (pi - pp) ** 2) + jnp.sum(wn * (ni - npp) ** 2)
    d_users = jnp.sum(wu * (ui - up) ** 2)
    discrepancy = d_items / (n_item * _D) + d_users / (n_user * _D)

    n = _B * n_item

    def kl(u, wsum_i, gu, m_i):
        s1 = jnp.sum(jnp.sum(u, axis=0, keepdims=True) * wsum_i)
        s2 = jnp.sum(gu * m_i)
        mean = s1 / n
        var = (s2 - s1 * s1 / n) / (n - 1.0)
        return -0.5 * jnp.log(var) + (var + mean * mean) / 2.0 - 0.5

    si_int = (jnp.sum(wp * pi, axis=0, keepdims=True)
              + jnp.sum(wn * ni, axis=0, keepdims=True))
    si_pop = (jnp.sum(wp * pp, axis=0, keepdims=True)
              + jnp.sum(wn * npp, axis=0, keepdims=True))
    gu_int = _gram(ui, ui)
    gu_pop = _gram(up, up)
    m_int = _gram(wp * pi, pi) + _gram(wn * ni, ni)
    m_pop = _gram(wp * pp, pp) + _gram(wn * npp, npp)
    kl_int = kl(ui, si_int, gu_int, m_int)
    kl_pop = kl(up, si_pop, gu_pop, m_pop)

    loss = (_INT_W * loss_int + _POP_W * loss_pop + loss_total
            - _DIS_PEN * discrepancy + _KL_W * (kl_int + kl_pop))
    out_ref[:, :] = jnp.full((1, 1), loss, dtype=jnp.float32)


_tc_loss = pl.pallas_call(
    _tc_loss_body,
    out_shape=jax.ShapeDtypeStruct((1, 1), jnp.float32),
)


def kernel(users_int_w, users_pop_w, items_int_w, items_pop_w,
           user, item_p, item_n, mask):
    uidx = user.reshape(-1).astype(jnp.int32)
    pidx = item_p.reshape(-1).astype(jnp.int32)
    nidx = item_n.reshape(-1).astype(jnp.int32)
    u_int, u_pop, p_int, p_pop, n_int, n_pop = _sc_gather()(
        users_int_w, users_pop_w, items_int_w, items_pop_w, uidx, pidx, nidx)
    iidx = jnp.concatenate([pidx, nidx])
    wi, wu = _tc_counts(iidx.reshape(1, -1), iidx.reshape(-1, 1),
                        uidx.reshape(1, -1), uidx.reshape(-1, 1))
    out = _tc_loss(u_int, u_pop, p_int, p_pop, n_int, n_pop,
                   wi, wu, mask.astype(jnp.float32))
    return out.reshape(())
